# baseline (speedup 1.0000x reference)
import jax
import jax.numpy as jnp
from jax import lax
from jax.experimental import pallas as pl
from jax.experimental.pallas import tpu as pltpu
from jax.experimental.pallas import tpu_sc as plsc

_BATCH = 16384
_XDIM = 128
_EDIM = 10
_ODIM = _XDIM + _EDIM
_MAXP = 36
_NC, _NS = 2, 16
_NW = _NC * _NS
_BPW = _BATCH // _NW
_ICHUNK = 128
_NPC = 2 * _BPW // _ICHUNK
_GRAN = 16
_NBLK = _BATCH * _ODIM // _GRAN
_XBLK = 512


def _sc_body(gidx_hbm, sidx_hbm, tbl_hbm, out16_hbm,
             gidx_v, sidx_v, piece_v, gsem, ssem):
    wid = lax.axis_index("s") * _NC + lax.axis_index("c")

    pltpu.sync_copy(gidx_hbm.at[pl.ds(wid * _NPC, _NPC)], gidx_v)
    pltpu.sync_copy(sidx_hbm.at[pl.ds(wid * _NPC, _NPC)], sidx_v)

    gathers = [
        pltpu.async_copy(tbl_hbm.at[gidx_v.at[pc]], piece_v.at[pc], gsem)
        for pc in range(_NPC)
    ]
    scats = []
    for pc in range(_NPC):
        gathers[pc].wait()
        scats.append(pltpu.async_copy(
            piece_v.at[pc], out16_hbm.at[sidx_v.at[pc]], ssem))
    for c in scats:
        c.wait()


def _xcopy_body(y_ref, x_ref, o_ref):
    o_ref[...] = x_ref[...]


def _build_patterns(emb):
    c = jnp.arange(8)[:, None]
    col = jnp.arange(32)[None, :]
    sel = (col >= 2 * c) & (col < 2 * c + 10)
    src = jnp.clip(col - 2 * c, 0, _EDIM - 1)
    pat = jnp.where(sel[None], emb[:, src], 0.0)
    return pat.reshape(_MAXP * 8 * 2, _GRAN)


def kernel(x, position, emb):
    pos = position.astype(jnp.int32)
    r = jnp.arange(_BATCH, dtype=jnp.int32)
    o = (10 * r) % _GRAN
    b = (_ODIM * r + _XDIM) // _GRAN
    double = (o > 6).astype(jnp.int32)
    gidx0 = (pos * 8 + o // 2) * 2
    gidx = jnp.stack([gidx0, gidx0 + double], axis=1).reshape(
        _NPC * _NW, _ICHUNK)
    sidx = jnp.stack([b, b + double], axis=1).reshape(_NPC * _NW, _ICHUNK)
    tbl = _build_patterns(emb)

    mesh = plsc.VectorSubcoreMesh(core_axis_name="c", subcore_axis_name="s")
    stripe = pl.kernel(
        _sc_body,
        out_type=jax.ShapeDtypeStruct((_NBLK, _GRAN), jnp.float32),
        mesh=mesh,
        scratch_types=[
            pltpu.VMEM((_NPC, _ICHUNK), jnp.int32),
            pltpu.VMEM((_NPC, _ICHUNK), jnp.int32),
            pltpu.VMEM((_NPC, _ICHUNK, _GRAN), jnp.float32),
            pltpu.SemaphoreType.DMA,
            pltpu.SemaphoreType.DMA,
        ],
        compiler_params=pltpu.CompilerParams(use_tc_tiling_on_sc=False),
    )(gidx, sidx, tbl)

    y0 = stripe.reshape(_BATCH, _ODIM)
    return pl.pallas_call(
        _xcopy_body,
        grid=(_BATCH // _XBLK,),
        in_specs=[
            pl.BlockSpec(memory_space=pl.ANY),
            pl.BlockSpec((_XBLK, _XDIM), lambda i: (i, 0)),
        ],
        out_specs=pl.BlockSpec((_XBLK, _XDIM), lambda i: (i, 0)),
        out_shape=jax.ShapeDtypeStruct((_BATCH, _ODIM), jnp.float32),
        input_output_aliases={0: 0},
    )(y0, x)


# speedup vs baseline: 1.3629x; 1.3629x over previous
import jax
import jax.numpy as jnp
from jax import lax
from jax.experimental import pallas as pl
from jax.experimental.pallas import tpu as pltpu
from jax.experimental.pallas import tpu_sc as plsc

_BATCH = 16384
_XDIM = 128
_EDIM = 10
_ODIM = _XDIM + _EDIM
_MAXP = 36
_PAD = 16
_NC, _NS = 2, 16
_NW = _NC * _NS
_BPW = _BATCH // _NW
_ICHUNK = 128
_NPC = _BPW // _ICHUNK
_XBLK = 512


def _sc_gather_body(idx_hbm, tbl_hbm, out_hbm, idx_v, piece_v, gsem):
    wid = lax.axis_index("s") * _NC + lax.axis_index("c")

    pltpu.sync_copy(idx_hbm.at[pl.ds(wid * _NPC, _NPC)], idx_v)

    gathers = [
        pltpu.async_copy(tbl_hbm.at[idx_v.at[pc]], piece_v.at[pc], gsem)
        for pc in range(_NPC)
    ]
    for pc in range(_NPC):
        gathers[pc].wait()
        pltpu.sync_copy(
            piece_v.at[pc],
            out_hbm.at[pl.ds(wid * _BPW + pc * _ICHUNK, _ICHUNK)])


def _concat_body(x_ref, pe_ref, o_ref):
    o_ref[...] = jnp.concatenate(
        [x_ref[...], pe_ref[:, :_EDIM]], axis=1)


def kernel(x, position, emb):
    idx = position.astype(jnp.int32).reshape(_NW * _NPC, _ICHUNK)
    tbl = jnp.pad(emb, ((0, 0), (0, _PAD - _EDIM)))

    mesh = plsc.VectorSubcoreMesh(core_axis_name="c", subcore_axis_name="s")
    pe16 = pl.kernel(
        _sc_gather_body,
        out_type=jax.ShapeDtypeStruct((_BATCH, _PAD), jnp.float32),
        mesh=mesh,
        scratch_types=[
            pltpu.VMEM((_NPC, _ICHUNK), jnp.int32),
            pltpu.VMEM((_NPC, _ICHUNK, _PAD), jnp.float32),
            pltpu.SemaphoreType.DMA,
        ],
        compiler_params=pltpu.CompilerParams(use_tc_tiling_on_sc=False),
    )(idx, tbl)

    return pl.pallas_call(
        _concat_body,
        grid=(_BATCH // _XBLK,),
        in_specs=[
            pl.BlockSpec((_XBLK, _XDIM), lambda i: (i, 0)),
            pl.BlockSpec((_XBLK, _PAD), lambda i: (i, 0)),
        ],
        out_specs=pl.BlockSpec((_XBLK, _ODIM), lambda i: (i, 0)),
        out_shape=jax.ShapeDtypeStruct((_BATCH, _ODIM), jnp.float32),
    )(x, pe16)


# speedup vs baseline: 1.5782x; 1.1580x over previous
import jax
import jax.numpy as jnp
from jax import lax
from jax.experimental import pallas as pl
from jax.experimental.pallas import tpu as pltpu
from jax.experimental.pallas import tpu_sc as plsc

_BATCH = 16384
_XDIM = 128
_EDIM = 10
_ODIM = _XDIM + _EDIM
_MAXP = 36
_PAD = 16
_NC, _NS = 2, 16
_NW = _NC * _NS
_BPW = _BATCH // _NW
_ICHUNK = 128
_NPC = _BPW // _ICHUNK
_XBLK = 2048


def _sc_gather_body(idx_hbm, tbl_hbm, out_hbm, idx_v, piece_v, gsem):
    wid = lax.axis_index("s") * _NC + lax.axis_index("c")

    pltpu.sync_copy(idx_hbm.at[pl.ds(wid * _NPC, _NPC)], idx_v)

    gathers = [
        pltpu.async_copy(tbl_hbm.at[idx_v.at[pc]], piece_v.at[pc], gsem)
        for pc in range(_NPC)
    ]
    for pc in range(_NPC):
        gathers[pc].wait()
        pltpu.sync_copy(
            piece_v.at[pc],
            out_hbm.at[pl.ds(wid * _BPW + pc * _ICHUNK, _ICHUNK)])


def _concat_body(x_ref, pe_ref, o_ref):
    o_ref[...] = jnp.concatenate(
        [x_ref[...], pe_ref[:, :_EDIM]], axis=1)


def kernel(x, position, emb):
    idx = position.astype(jnp.int32).reshape(_NW * _NPC, _ICHUNK)
    tbl = jnp.pad(emb, ((0, 0), (0, _PAD - _EDIM)))

    mesh = plsc.VectorSubcoreMesh(core_axis_name="c", subcore_axis_name="s")
    pe16 = pl.kernel(
        _sc_gather_body,
        out_type=jax.ShapeDtypeStruct((_BATCH, _PAD), jnp.float32),
        mesh=mesh,
        scratch_types=[
            pltpu.VMEM((_NPC, _ICHUNK), jnp.int32),
            pltpu.VMEM((_NPC, _ICHUNK, _PAD), jnp.float32),
            pltpu.SemaphoreType.DMA,
        ],
        compiler_params=pltpu.CompilerParams(use_tc_tiling_on_sc=False),
    )(idx, tbl)

    return pl.pallas_call(
        _concat_body,
        grid=(_BATCH // _XBLK,),
        in_specs=[
            pl.BlockSpec((_XBLK, _XDIM), lambda i: (i, 0)),
            pl.BlockSpec((_XBLK, _PAD), lambda i: (i, 0)),
        ],
        out_specs=pl.BlockSpec((_XBLK, _ODIM), lambda i: (i, 0)),
        out_shape=jax.ShapeDtypeStruct((_BATCH, _ODIM), jnp.float32),
    )(x, pe16)


# speedup vs baseline: 1.6062x; 1.0177x over previous
import jax
import jax.numpy as jnp
from jax import lax
from jax.experimental import pallas as pl
from jax.experimental.pallas import tpu as pltpu
from jax.experimental.pallas import tpu_sc as plsc

_BATCH = 16384
_XDIM = 128
_EDIM = 10
_ODIM = _XDIM + _EDIM
_MAXP = 36
_PAD = 16
_NC, _NS = 2, 16
_NW = _NC * _NS
_BPW = _BATCH // _NW
_ICHUNK = 128
_NPC = _BPW // _ICHUNK
_XBLK = 4096


def _sc_gather_body(idx_hbm, tbl_hbm, out_hbm, idx_v, piece_v, gsem):
    wid = lax.axis_index("s") * _NC + lax.axis_index("c")

    pltpu.sync_copy(idx_hbm.at[pl.ds(wid * _NPC, _NPC)], idx_v)

    gathers = [
        pltpu.async_copy(tbl_hbm.at[idx_v.at[pc]], piece_v.at[pc], gsem)
        for pc in range(_NPC)
    ]
    for pc in range(_NPC):
        gathers[pc].wait()
        pltpu.sync_copy(
            piece_v.at[pc],
            out_hbm.at[pl.ds(wid * _BPW + pc * _ICHUNK, _ICHUNK)])


def _concat_body(x_ref, pe_ref, o_ref):
    o_ref[...] = jnp.concatenate(
        [x_ref[...], pe_ref[:, :_EDIM]], axis=1)


def kernel(x, position, emb):
    idx = position.astype(jnp.int32).reshape(_NW * _NPC, _ICHUNK)
    tbl = jnp.pad(emb, ((0, 0), (0, _PAD - _EDIM)))

    mesh = plsc.VectorSubcoreMesh(core_axis_name="c", subcore_axis_name="s")
    pe16 = pl.kernel(
        _sc_gather_body,
        out_type=jax.ShapeDtypeStruct((_BATCH, _PAD), jnp.float32),
        mesh=mesh,
        scratch_types=[
            pltpu.VMEM((_NPC, _ICHUNK), jnp.int32),
            pltpu.VMEM((_NPC, _ICHUNK, _PAD), jnp.float32),
            pltpu.SemaphoreType.DMA,
        ],
        compiler_params=pltpu.CompilerParams(use_tc_tiling_on_sc=False),
    )(idx, tbl)

    return pl.pallas_call(
        _concat_body,
        grid=(_BATCH // _XBLK,),
        in_specs=[
            pl.BlockSpec((_XBLK, _XDIM), lambda i: (i, 0)),
            pl.BlockSpec((_XBLK, _PAD), lambda i: (i, 0)),
        ],
        out_specs=pl.BlockSpec((_XBLK, _ODIM), lambda i: (i, 0)),
        out_shape=jax.ShapeDtypeStruct((_BATCH, _ODIM), jnp.float32),
    )(x, pe16)


# speedup vs baseline: 1.6297x; 1.0146x over previous
import jax
import jax.numpy as jnp
from jax import lax
from jax.experimental import pallas as pl
from jax.experimental.pallas import tpu as pltpu
from jax.experimental.pallas import tpu_sc as plsc

_BATCH = 16384
_XDIM = 128
_EDIM = 10
_ODIM = _XDIM + _EDIM
_MAXP = 36
_PAD = 16
_NC, _NS = 2, 16
_NW = _NC * _NS
_BPW = _BATCH // _NW
_ICHUNK = 128
_NPC = _BPW // _ICHUNK
_XBLK = 8192


def _sc_gather_body(idx_hbm, tbl_hbm, out_hbm, idx_v, piece_v, gsem):
    wid = lax.axis_index("s") * _NC + lax.axis_index("c")

    pltpu.sync_copy(idx_hbm.at[pl.ds(wid * _NPC, _NPC)], idx_v)

    gathers = [
        pltpu.async_copy(tbl_hbm.at[idx_v.at[pc]], piece_v.at[pc], gsem)
        for pc in range(_NPC)
    ]
    for pc in range(_NPC):
        gathers[pc].wait()
        pltpu.sync_copy(
            piece_v.at[pc],
            out_hbm.at[pl.ds(wid * _BPW + pc * _ICHUNK, _ICHUNK)])


def _concat_body(x_ref, pe_ref, o_ref):
    o_ref[...] = jnp.concatenate(
        [x_ref[...], pe_ref[:, :_EDIM]], axis=1)


def kernel(x, position, emb):
    idx = position.astype(jnp.int32).reshape(_NW * _NPC, _ICHUNK)
    tbl = jnp.pad(emb, ((0, 0), (0, _PAD - _EDIM)))

    mesh = plsc.VectorSubcoreMesh(core_axis_name="c", subcore_axis_name="s")
    pe16 = pl.kernel(
        _sc_gather_body,
        out_type=jax.ShapeDtypeStruct((_BATCH, _PAD), jnp.float32),
        mesh=mesh,
        scratch_types=[
            pltpu.VMEM((_NPC, _ICHUNK), jnp.int32),
            pltpu.VMEM((_NPC, _ICHUNK, _PAD), jnp.float32),
            pltpu.SemaphoreType.DMA,
        ],
        compiler_params=pltpu.CompilerParams(use_tc_tiling_on_sc=False),
    )(idx, tbl)

    return pl.pallas_call(
        _concat_body,
        grid=(_BATCH // _XBLK,),
        in_specs=[
            pl.BlockSpec((_XBLK, _XDIM), lambda i: (i, 0)),
            pl.BlockSpec((_XBLK, _PAD), lambda i: (i, 0)),
        ],
        out_specs=pl.BlockSpec((_XBLK, _ODIM), lambda i: (i, 0)),
        out_shape=jax.ShapeDtypeStruct((_BATCH, _ODIM), jnp.float32),
    )(x, pe16)
